# baseline (speedup 1.0000x reference)
import functools

import jax
import jax.numpy as jnp
from jax import lax
from jax.experimental import pallas as pl
from jax.experimental.pallas import tpu as pltpu
from jax.experimental.pallas import tpu_sc as plsc

NC = 2
NS = 16
NW = NC * NS
LANES = 16

DEG = 32
D = 128
B_PAD = 10240
B_PER_W = B_PAD // NW
CHUNK_DST = 4
CHUNK_ROWS = CHUNK_DST * DEG
N_CHUNKS = B_PER_W // CHUNK_DST
SELF_CHUNK = 40
N_SELF = B_PER_W // SELF_CHUNK


def _sc_gather_body(neigh_hbm, nodes_hbm, table_hbm,
                    self_out, nsum_out,
                    idx2d, sidx, rbuf0, rbuf1, sbuf, sbuf1, oslab,
                    sem0, sem1, ssem):
    wid = lax.axis_index("s") * NC + lax.axis_index("c")

    pltpu.sync_copy(neigh_hbm.at[pl.ds(wid * N_CHUNKS, N_CHUNKS)], idx2d)
    pltpu.sync_copy(nodes_hbm.at[pl.ds(wid * N_SELF, N_SELF)], sidx)

    n_c = D // LANES
    r_unroll = 8

    def reduce_chunk(rbuf, g):
        zero = jnp.zeros((LANES,), jnp.float32)
        def body(r, accs, rbuf=rbuf):
            base = r * r_unroll
            new = []
            for d in range(CHUNK_DST):
                for c in range(n_c):
                    v = accs[d * n_c + c]
                    for rr in range(r_unroll):
                        v = v + rbuf[d * DEG + base + rr, pl.ds(c * LANES, LANES)]
                    new.append(v)
            return tuple(new)
        accs = lax.fori_loop(0, DEG // r_unroll, body,
                             tuple([zero] * (CHUNK_DST * n_c)))
        for d in range(CHUNK_DST):
            for c in range(n_c):
                oslab[g * CHUNK_DST + d, pl.ds(c * LANES, LANES)] = accs[d * n_c + c]

    pltpu.async_copy(table_hbm.at[idx2d.at[0]], rbuf0, sem0)

    def outer(i, carry):
        g0 = 2 * i
        g1 = g0 + 1
        pltpu.async_copy(table_hbm.at[idx2d.at[g1]], rbuf1, sem1)
        pltpu.make_async_copy(table_hbm.at[idx2d.at[g0]], rbuf0, sem0).wait()
        reduce_chunk(rbuf0, g0)

        @pl.when(g0 + 2 < N_CHUNKS)
        def _():
            pltpu.async_copy(table_hbm.at[idx2d.at[g0 + 2]], rbuf0, sem0)

        pltpu.make_async_copy(table_hbm.at[idx2d.at[g1]], rbuf1, sem1).wait()
        reduce_chunk(rbuf1, g1)
        return carry

    lax.fori_loop(0, N_CHUNKS // 2, outer, 0)

    pltpu.sync_copy(oslab, nsum_out.at[pl.ds(wid * B_PER_W, B_PER_W)])

    sb = (sbuf, sbuf1)
    pltpu.async_copy(table_hbm.at[sidx.at[0]], sb[0], ssem)
    for j in range(N_SELF):
        if j + 1 < N_SELF:
            pltpu.async_copy(table_hbm.at[sidx.at[j + 1]], sb[(j + 1) % 2], ssem)
        pltpu.make_async_copy(table_hbm.at[sidx.at[j]], sb[j % 2], ssem).wait()
        pltpu.sync_copy(
            sb[j % 2],
            self_out.at[pl.ds(wid * B_PER_W + j * SELF_CHUNK, SELF_CHUNK)])


@jax.jit
def _sc_gather(neigh2d, nodes2d, table):
    mesh = plsc.VectorSubcoreMesh(core_axis_name="c", subcore_axis_name="s",
                                  num_cores=NC, num_subcores=NS)
    fn = functools.partial(
        pl.kernel,
        out_type=(
            jax.ShapeDtypeStruct((B_PAD, D), jnp.float32),
            jax.ShapeDtypeStruct((B_PAD, D), jnp.float32),
        ),
        mesh=mesh,
        scratch_types=[
            pltpu.VMEM((N_CHUNKS, CHUNK_ROWS), jnp.int32),
            pltpu.VMEM((N_SELF, SELF_CHUNK), jnp.int32),
            pltpu.VMEM((CHUNK_ROWS, D), jnp.float32),
            pltpu.VMEM((CHUNK_ROWS, D), jnp.float32),
            pltpu.VMEM((SELF_CHUNK, D), jnp.float32),
            pltpu.VMEM((SELF_CHUNK, D), jnp.float32),
            pltpu.VMEM((B_PER_W, D), jnp.float32),
            pltpu.SemaphoreType.DMA,
            pltpu.SemaphoreType.DMA,
            pltpu.SemaphoreType.DMA,
        ],
    )(_sc_gather_body)
    return fn(neigh2d, nodes2d, table)


def _mm_body(self_ref, nsum_ref, wa_ref, wb_ref, b_ref, o_ref):
    x = (jnp.dot(self_ref[...], wa_ref[...], preferred_element_type=jnp.float32)
         + jnp.dot(nsum_ref[...], wb_ref[...], preferred_element_type=jnp.float32)
         + b_ref[...])
    o_ref[...] = jnp.maximum(x, 0.0)


def _combine(self_rows, nsum, wa, wb_scaled, b2d):
    blk = 1024
    return pl.pallas_call(
        _mm_body,
        grid=(B_PAD // blk,),
        in_specs=[
            pl.BlockSpec((blk, D), lambda i: (i, 0)),
            pl.BlockSpec((blk, D), lambda i: (i, 0)),
            pl.BlockSpec((D, D), lambda i: (0, 0)),
            pl.BlockSpec((D, D), lambda i: (0, 0)),
            pl.BlockSpec((1, D), lambda i: (0, 0)),
        ],
        out_specs=pl.BlockSpec((blk, D), lambda i: (i, 0)),
        out_shape=jax.ShapeDtypeStruct((B_PAD, D), jnp.float32),
    )(self_rows, nsum, wa, wb_scaled, b2d)


def kernel(nodes, neighbors, table, W1, b1):
    B = nodes.shape[0]
    pad = B_PAD - B
    nodes_p = jnp.pad(nodes, (0, pad)).reshape(B_PAD // SELF_CHUNK, SELF_CHUNK)
    neigh_p = jnp.pad(neighbors, ((0, pad), (0, 0))).reshape(
        B_PAD * DEG // CHUNK_ROWS, CHUNK_ROWS)
    self_rows, nsum = _sc_gather(neigh_p, nodes_p, table)
    wa = W1[:D]
    wb_scaled = W1[D:] * (1.0 / DEG)
    out = _combine(self_rows, nsum, wa, wb_scaled, b1.reshape(1, D))
    return out[:B]


# speedup vs baseline: 2.3110x; 2.3110x over previous
import functools

import jax
import jax.numpy as jnp
from jax import lax
from jax.experimental import pallas as pl
from jax.experimental.pallas import tpu as pltpu
from jax.experimental.pallas import tpu_sc as plsc

NC = 2
NS = 16
NW = NC * NS
LANES = 16

DEG = 32
D = 128
B_PAD = 10240
B_PER_W = B_PAD // NW
CHUNK_DST = 4
CHUNK_ROWS = CHUNK_DST * DEG
N_CHUNKS = B_PER_W // CHUNK_DST
SELF_CHUNK = 40
N_SELF = B_PER_W // SELF_CHUNK


def _sc_gather_body(neigh_hbm, nodes_hbm, table_hbm,
                    self_out, nsum_out,
                    idx2d, sidx, rbuf0, rbuf1, sbuf, sbuf1, oslab,
                    sem0, sem1, ssem):
    wid = lax.axis_index("s") * NC + lax.axis_index("c")

    pltpu.sync_copy(neigh_hbm.at[pl.ds(wid * N_CHUNKS, N_CHUNKS)], idx2d)
    pltpu.sync_copy(nodes_hbm.at[pl.ds(wid * N_SELF, N_SELF)], sidx)

    n_c = D // LANES
    r_unroll = 8

    def reduce_chunk(rbuf, g):
        zero = jnp.zeros((LANES,), jnp.float32)
        def body(r, accs, rbuf=rbuf):
            base = r * r_unroll
            new = []
            for d in range(CHUNK_DST):
                for c in range(n_c):
                    v = accs[d * n_c + c]
                    for rr in range(r_unroll):
                        v = v + rbuf[d * DEG + base + rr, pl.ds(c * LANES, LANES)]
                    new.append(v)
            return tuple(new)
        accs = lax.fori_loop(0, DEG // r_unroll, body,
                             tuple([zero] * (CHUNK_DST * n_c)))
        for d in range(CHUNK_DST):
            for c in range(n_c):
                oslab[g * CHUNK_DST + d, pl.ds(c * LANES, LANES)] = accs[d * n_c + c]

    pltpu.async_copy(table_hbm.at[idx2d.at[0]], rbuf0, sem0)

    def outer(i, carry):
        g0 = 2 * i
        g1 = g0 + 1
        pltpu.async_copy(table_hbm.at[idx2d.at[g1]], rbuf1, sem1)
        pltpu.make_async_copy(table_hbm.at[idx2d.at[g0]], rbuf0, sem0).wait()
        reduce_chunk(rbuf0, g0)

        @pl.when(g0 + 2 < N_CHUNKS)
        def _():
            pltpu.async_copy(table_hbm.at[idx2d.at[g0 + 2]], rbuf0, sem0)

        pltpu.make_async_copy(table_hbm.at[idx2d.at[g1]], rbuf1, sem1).wait()
        reduce_chunk(rbuf1, g1)
        return carry

    lax.fori_loop(0, N_CHUNKS // 2, outer, 0)

    pltpu.sync_copy(oslab, nsum_out.at[pl.ds(wid * B_PER_W, B_PER_W)])

    sb = (sbuf, sbuf1)
    pltpu.async_copy(table_hbm.at[sidx.at[0]], sb[0], ssem)
    for j in range(N_SELF):
        if j + 1 < N_SELF:
            pltpu.async_copy(table_hbm.at[sidx.at[j + 1]], sb[(j + 1) % 2], ssem)
        pltpu.make_async_copy(table_hbm.at[sidx.at[j]], sb[j % 2], ssem).wait()
        pltpu.sync_copy(
            sb[j % 2],
            self_out.at[pl.ds(wid * B_PER_W + j * SELF_CHUNK, SELF_CHUNK)])


@jax.jit
def _sc_gather(neigh2d, nodes2d, table):
    mesh = plsc.VectorSubcoreMesh(core_axis_name="c", subcore_axis_name="s",
                                  num_cores=NC, num_subcores=NS)
    fn = functools.partial(
        pl.kernel,
        out_type=(
            jax.ShapeDtypeStruct((B_PAD, D), jnp.float32),
            jax.ShapeDtypeStruct((B_PAD, D), jnp.float32),
        ),
        mesh=mesh,
        scratch_types=[
            pltpu.VMEM((N_CHUNKS, CHUNK_ROWS), jnp.int32),
            pltpu.VMEM((N_SELF, SELF_CHUNK), jnp.int32),
            pltpu.VMEM((CHUNK_ROWS, D), jnp.float32),
            pltpu.VMEM((CHUNK_ROWS, D), jnp.float32),
            pltpu.VMEM((SELF_CHUNK, D), jnp.float32),
            pltpu.VMEM((SELF_CHUNK, D), jnp.float32),
            pltpu.VMEM((B_PER_W, D), jnp.float32),
            pltpu.SemaphoreType.DMA,
            pltpu.SemaphoreType.DMA,
            pltpu.SemaphoreType.DMA,
        ],
    )(_sc_gather_body)
    return fn(neigh2d, nodes2d, table)


def _mm_body(self_ref, nsum_ref, wa_ref, wb_ref, b_ref, o_ref):
    x = (jnp.dot(self_ref[...], wa_ref[...], preferred_element_type=jnp.float32)
         + jnp.dot(nsum_ref[...], wb_ref[...], preferred_element_type=jnp.float32)
         + b_ref[...])
    o_ref[...] = jnp.maximum(x, 0.0)


def _combine(self_rows, nsum, wa, wb_scaled, b2d):
    blk = 1024
    return pl.pallas_call(
        _mm_body,
        grid=(B_PAD // blk,),
        in_specs=[
            pl.BlockSpec((blk, D), lambda i: (i, 0)),
            pl.BlockSpec((blk, D), lambda i: (i, 0)),
            pl.BlockSpec((D, D), lambda i: (0, 0)),
            pl.BlockSpec((D, D), lambda i: (0, 0)),
            pl.BlockSpec((1, D), lambda i: (0, 0)),
        ],
        out_specs=pl.BlockSpec((blk, D), lambda i: (i, 0)),
        out_shape=jax.ShapeDtypeStruct((B_PAD, D), jnp.float32),
    )(self_rows, nsum, wa, wb_scaled, b2d)


def kernel(nodes, neighbors, table, W1, b1):
    B = nodes.shape[0]
    pad = B_PAD - B
    n_nodes = table.shape[0]
    pad_nodes = (jnp.arange(pad, dtype=jnp.int32) * 131) % n_nodes
    pad_neigh = ((jnp.arange(pad * DEG, dtype=jnp.int32) * 131) % n_nodes
                 ).reshape(pad, DEG)
    nodes_p = jnp.concatenate([nodes, pad_nodes]).reshape(
        B_PAD // SELF_CHUNK, SELF_CHUNK)
    neigh_p = jnp.concatenate([neighbors, pad_neigh], axis=0).reshape(
        B_PAD * DEG // CHUNK_ROWS, CHUNK_ROWS)
    self_rows, nsum = _sc_gather(neigh_p, nodes_p, table)
    wa = W1[:D]
    wb_scaled = W1[D:] * (1.0 / DEG)
    out = _combine(self_rows, nsum, wa, wb_scaled, b1.reshape(1, D))
    return out[:B]
